# baseline (speedup 1.0000x reference)
import functools

import jax
import jax.numpy as jnp
import numpy as np
from jax import lax
from jax.experimental import pallas as pl
from jax.experimental.pallas import tpu as pltpu
from jax.experimental.pallas import tpu_sc as plsc

N = 10000
E = 320000
D = 128
H = 8
HD = 16

NC = 2
NS = 16
NW = NC * NS

BLK = 2000
NBLK = E // BLK

CH = 40


def _mesh():
    return plsc.VectorSubcoreMesh(core_axis_name="c", subcore_axis_name="s")


def _qkv_body(x_ref, w_ref, b_ref, q_ref, k_ref, v_ref):
    y = jnp.dot(x_ref[...], w_ref[...], preferred_element_type=jnp.float32)
    y = y + b_ref[...]
    q_ref[...] = y[:, :D]
    k_ref[...] = y[:, D:2 * D]
    v_ref[...] = y[:, 2 * D:]


def _qkv(x, wcat, bcat):
    return pl.pallas_call(
        _qkv_body,
        out_shape=[jax.ShapeDtypeStruct((N, D), jnp.float32)] * 3,
    )(x, wcat, bcat)


def _score1_sc(kh, qh, src, dst):
    EP = E // NW
    NB = 5
    NG = EP // CH // NB
    assert NG * NB * CH == EP

    scratch = []
    scratch += [pltpu.VMEM((CH,), jnp.int32) for _ in range(NB)]
    scratch += [pltpu.VMEM((CH,), jnp.int32) for _ in range(NB)]
    scratch += [pltpu.VMEM((CH, D), jnp.float32) for _ in range(NB)]
    scratch += [pltpu.VMEM((CH, D), jnp.float32) for _ in range(NB)]
    scratch += [pltpu.SemaphoreType.DMA for _ in range(3 * NB)]

    @functools.partial(
        pl.kernel,
        mesh=_mesh(),
        out_type=jax.ShapeDtypeStruct((E, D), jnp.float32),
        scratch_types=scratch,
    )
    def k(kh_hbm, qh_hbm, src_hbm, dst_hbm, out_hbm, *refs):
        nb = NB
        isv = refs[0:nb]
        idv = refs[nb:2 * nb]
        bk = refs[2 * nb:3 * nb]
        bq = refs[3 * nb:4 * nb]
        sem_l = refs[4 * nb:5 * nb]
        sem_g = refs[5 * nb:6 * nb]
        sem_o = refs[6 * nb:7 * nb]

        wid = lax.axis_index("s") * NC + lax.axis_index("c")
        tbase = wid * EP

        def issue_lin(g, b):
            sl = pl.ds(tbase + (g * nb + b) * CH, CH)
            pltpu.async_copy(src_hbm.at[sl], isv[b], sem_l[b])
            pltpu.async_copy(dst_hbm.at[sl], idv[b], sem_l[b])

        def wait_lin(b):
            pltpu.make_async_copy(src_hbm.at[pl.ds(0, CH)], isv[b],
                                  sem_l[b]).wait()
            pltpu.make_async_copy(dst_hbm.at[pl.ds(0, CH)], idv[b],
                                  sem_l[b]).wait()

        def group(g, carry):
            for b in range(nb):
                wait_lin(b)
                pltpu.async_copy(kh_hbm.at[isv[b]], bk[b], sem_g[b])
                pltpu.async_copy(qh_hbm.at[idv[b]], bq[b], sem_g[b])
            for b in range(nb):
                pltpu.make_async_copy(kh_hbm.at[isv[b]], bk[b],
                                      sem_g[b]).wait()
                pltpu.make_async_copy(qh_hbm.at[idv[b]], bq[b],
                                      sem_g[b]).wait()

                def row(i, carry2, b=b):
                    for j in (2 * i, 2 * i + 1):
                        for h in range(H):
                            sl = pl.ds(h * HD, HD)
                            bk[b][j, sl] = bk[b][j, sl] + bq[b][j, sl]
                    return carry2

                lax.fori_loop(0, CH // 2, row, 0)
                out_sl = pl.ds(tbase + (g * nb + b) * CH, CH)
                pltpu.async_copy(bk[b], out_hbm.at[out_sl], sem_o[b])
            for b in range(nb):
                pltpu.make_async_copy(bk[b], out_hbm.at[pl.ds(0, CH)],
                                      sem_o[b]).wait()

                @pl.when(g < NG - 1)
                def _(b=b):
                    issue_lin(g + 1, b)

            return carry

        for b in range(nb):
            issue_lin(0, b)
        lax.fori_loop(0, NG, group, 0)

    return k(kh, qh, src, dst)


def _edge_body(ea_ref, s1_ref, ew_ref, eb_ref, a_ref, exp_ref, eow_ref,
               eob_ref, etw_ref, w_ref, w16_ref, ehp_ref, sums_ref):
    i = pl.program_id(0)
    ea = ea_ref[...]
    ef = jnp.dot(ea.astype(jnp.bfloat16), ew_ref[...],
                 preferred_element_type=jnp.float32)
    ef = ef + eb_ref[...]
    s2 = ef[:, :D] * ef[:, D:]
    pn = jnp.sign(s2) * jnp.sqrt(jnp.abs(s2))
    et = jax.nn.relu(s1_ref[...] + pn)
    etb = et.astype(jnp.bfloat16)
    sc = jnp.dot(etb, a_ref[...], preferred_element_type=jnp.float32)
    w16 = jnp.exp(jnp.clip(sc, -5.0, 5.0))
    w16_ref[...] = w16
    wx = jnp.dot(w16, exp_ref[...], preferred_element_type=jnp.float32)
    w_ref[...] = wx
    etw_ref[...] = et * wx
    ehp = ea + jnp.dot(etb, eow_ref[...], preferred_element_type=jnp.float32)
    ehp = ehp + eob_ref[...]
    ehp_ref[...] = ehp

    @pl.when(i == 0)
    def _():
        sums_ref[...] = jnp.zeros_like(sums_ref)

    sums_ref[0:1, :] = sums_ref[0:1, :] + jnp.sum(ehp, axis=0, keepdims=True)
    sums_ref[1:2, :] = sums_ref[1:2, :] + jnp.sum(ehp * ehp, axis=0,
                                                  keepdims=True)


def _edge_tc(edge_attr, score1, ewp, ebp, a128, expm, eow, eob):
    return pl.pallas_call(
        _edge_body,
        grid=(NBLK,),
        in_specs=[
            pl.BlockSpec((BLK, D), lambda i: (i, 0)),
            pl.BlockSpec((BLK, D), lambda i: (i, 0)),
            pl.BlockSpec((D, 2 * D), lambda i: (0, 0)),
            pl.BlockSpec((1, 2 * D), lambda i: (0, 0)),
            pl.BlockSpec((D, HD), lambda i: (0, 0)),
            pl.BlockSpec((HD, D), lambda i: (0, 0)),
            pl.BlockSpec((D, D), lambda i: (0, 0)),
            pl.BlockSpec((1, D), lambda i: (0, 0)),
        ],
        out_specs=[
            pl.BlockSpec((BLK, D), lambda i: (i, 0)),
            pl.BlockSpec((BLK, D), lambda i: (i, 0)),
            pl.BlockSpec((BLK, HD), lambda i: (i, 0)),
            pl.BlockSpec((BLK, D), lambda i: (i, 0)),
            pl.BlockSpec((8, D), lambda i: (0, 0)),
        ],
        out_shape=[
            jax.ShapeDtypeStruct((E, D), jnp.float32),
            jax.ShapeDtypeStruct((E, D), jnp.float32),
            jax.ShapeDtypeStruct((E, HD), jnp.float32),
            jax.ShapeDtypeStruct((E, D), jnp.float32),
            jax.ShapeDtypeStruct((8, D), jnp.float32),
        ],
    )(edge_attr, score1, ewp, ebp, a128, expm, eow, eob)


def _stab_sc(w, dst, zeros128):
    EP = E // NW
    NB = 2
    CHS = 40
    NG = EP // CHS // NB
    assert NG * NB * CHS == EP
    STRIPE = (N // NS) // 8 * 8
    SREM = N - STRIPE * NS

    scratch = [pltpu.VMEM_SHARED((N, D), jnp.float32)]
    scratch += [pltpu.VMEM((CHS,), jnp.int32) for _ in range(NB)]
    scratch += [pltpu.VMEM((CHS, D), jnp.float32) for _ in range(NB)]
    scratch += [pltpu.SemaphoreType.DMA for _ in range(2 * NB)]

    @functools.partial(
        pl.kernel,
        mesh=_mesh(),
        out_type=jax.ShapeDtypeStruct((NC, N, D), jnp.float32),
        scratch_types=scratch,
    )
    def k(w_hbm, dst_hbm, z_hbm, out_hbm, *refs):
        nb = NB
        stab = refs[0]
        idv = refs[1:1 + nb]
        wb = refs[1 + nb:1 + 2 * nb]
        sem_l = refs[1 + 2 * nb:1 + 3 * nb]
        sem_s = refs[1 + 3 * nb:1 + 4 * nb]

        c = lax.axis_index("c")
        sid = lax.axis_index("s")
        tbase = c * (E // NC) + sid * EP

        rows = pl.ds(sid * STRIPE, STRIPE)
        rrem = pl.ds(NS * STRIPE, SREM)
        pltpu.sync_copy(z_hbm.at[rows], stab.at[rows])

        @pl.when(sid == NS - 1)
        def _():
            pltpu.sync_copy(z_hbm.at[rrem], stab.at[rrem])

        plsc.subcore_barrier()

        def issue_lin(g, b):
            sl = pl.ds(tbase + (g * nb + b) * CHS, CHS)
            pltpu.async_copy(dst_hbm.at[sl], idv[b], sem_l[b])
            pltpu.async_copy(w_hbm.at[sl], wb[b], sem_l[b])

        def group(g, carry):
            for b in range(nb):
                pltpu.make_async_copy(dst_hbm.at[pl.ds(0, CHS)], idv[b],
                                      sem_l[b]).wait()
                pltpu.make_async_copy(w_hbm.at[pl.ds(0, CHS)], wb[b],
                                      sem_l[b]).wait()
                pltpu.async_copy(wb[b], stab.at[idv[b]], sem_s[b], add=True)
            for b in range(nb):
                pltpu.make_async_copy(wb[b], stab.at[idv[b]],
                                      sem_s[b]).wait()

                @pl.when(g < NG - 1)
                def _(b=b):
                    issue_lin(g + 1, b)

            return carry

        for b in range(nb):
            issue_lin(0, b)
        lax.fori_loop(0, NG, group, 0)

        plsc.subcore_barrier()
        pltpu.sync_copy(stab.at[rows], out_hbm.at[c, rows])

        @pl.when(sid == NS - 1)
        def _():
            pltpu.sync_copy(stab.at[rrem], out_hbm.at[c, rrem])

    return k(w, dst, zeros128)


AGG_NB = 4


def _agg_sc(vh, etw, w16, src, dst, zeros128):
    EP = E // NS
    NCH = EP // CH
    NG = NCH // AGG_NB
    TAIL = EP - NG * AGG_NB * CH
    STRIPE = (N // NS) // 8 * 8
    SREM = N - STRIPE * NS

    assert TAIL == 0
    scratch = [pltpu.VMEM_SHARED((N, D), jnp.float32)]
    scratch += [pltpu.VMEM((CH,), jnp.int32) for _ in range(AGG_NB)]
    scratch += [pltpu.VMEM((CH,), jnp.int32) for _ in range(AGG_NB)]
    scratch += [pltpu.VMEM((CH, D), jnp.float32) for _ in range(AGG_NB)]
    scratch += [pltpu.VMEM((CH, HD), jnp.float32) for _ in range(AGG_NB)]
    scratch += [pltpu.SemaphoreType.DMA for _ in range(3 * AGG_NB)]

    @functools.partial(
        pl.kernel,
        mesh=_mesh(),
        out_type=jax.ShapeDtypeStruct((NC, N, D), jnp.float32),
        scratch_types=scratch,
    )
    def k(vh_hbm, etw_hbm, w_hbm, src_hbm, dst_hbm, z_hbm, out_hbm, *refs):
        nb = AGG_NB
        tab = refs[0]
        isv = refs[1:1 + nb]
        idv = refs[1 + nb:1 + 2 * nb]
        pb = refs[1 + 2 * nb:1 + 3 * nb]
        wb = refs[1 + 3 * nb:1 + 4 * nb]
        sem_l = refs[1 + 4 * nb:1 + 5 * nb]
        sem_g = refs[1 + 5 * nb:1 + 6 * nb]
        sem_s = refs[1 + 6 * nb:1 + 7 * nb]

        c = lax.axis_index("c")
        sid = lax.axis_index("s")
        tbase = sid * EP

        rows = pl.ds(sid * STRIPE, STRIPE)
        rrem = pl.ds(NS * STRIPE, SREM)
        pltpu.sync_copy(z_hbm.at[rows], tab.at[rows])

        @pl.when(sid == NS - 1)
        def _():
            pltpu.sync_copy(z_hbm.at[rrem], tab.at[rrem])

        plsc.subcore_barrier()

        def issue_lin(g, b):
            base = tbase + (g * nb + b) * CH
            sl = pl.ds(base, CH)
            pltpu.async_copy(dst_hbm.at[sl], idv[b], sem_l[b])

            @pl.when(c == 0)
            def _():
                pltpu.async_copy(w_hbm.at[sl], wb[b], sem_l[b])
                pltpu.async_copy(src_hbm.at[sl], isv[b], sem_l[b])

            @pl.when(c != 0)
            def _():
                pltpu.async_copy(etw_hbm.at[sl], pb[b], sem_l[b])

        def wait_lin(b):
            pltpu.make_async_copy(dst_hbm.at[pl.ds(0, CH)], idv[b],
                                  sem_l[b]).wait()

            @pl.when(c == 0)
            def _():
                pltpu.make_async_copy(w_hbm.at[pl.ds(0, CH)], wb[b],
                                      sem_l[b]).wait()
                pltpu.make_async_copy(src_hbm.at[pl.ds(0, CH)], isv[b],
                                      sem_l[b]).wait()

            @pl.when(c != 0)
            def _():
                pltpu.make_async_copy(etw_hbm.at[pl.ds(0, CH)], pb[b],
                                      sem_l[b]).wait()

        def group(g, carry):
            @pl.when(c == 0)
            def _():
                for b in range(nb):
                    wait_lin(b)
                    pltpu.async_copy(vh_hbm.at[isv[b]], pb[b], sem_g[b])
                for b in range(nb):
                    pltpu.make_async_copy(vh_hbm.at[isv[b]], pb[b],
                                          sem_g[b]).wait()

                    def row(i, carry2, b=b):
                        for j in (2 * i, 2 * i + 1):
                            wv = wb[b][j, :]
                            for h in range(H):
                                av = jnp.full((HD,), wv[h],
                                              dtype=jnp.float32)
                                sl = pl.ds(h * HD, HD)
                                pb[b][j, sl] = pb[b][j, sl] * av
                        return carry2

                    lax.fori_loop(0, CH // 2, row, 0)
                    pltpu.async_copy(pb[b], tab.at[idv[b]], sem_s[b],
                                     add=True)

            @pl.when(c != 0)
            def _():
                for b in range(nb):
                    wait_lin(b)
                    pltpu.async_copy(pb[b], tab.at[idv[b]], sem_s[b],
                                     add=True)

            for b in range(nb):
                pltpu.make_async_copy(pb[b], tab.at[idv[b]], sem_s[b]).wait()

                @pl.when(g < NG - 1)
                def _(b=b):
                    issue_lin(g + 1, b)

            return carry

        for b in range(nb):
            issue_lin(0, b)
        lax.fori_loop(0, NG, group, 0)

        plsc.subcore_barrier()
        pltpu.sync_copy(tab.at[rows], out_hbm.at[c, rows])

        @pl.when(sid == NS - 1)
        def _():
            pltpu.sync_copy(tab.at[rrem], out_hbm.at[c, rrem])

    return k(vh, etw, w16, src, dst, zeros128)


def _node_body(ov_ref, s_ref, x_ref, ld_ref, w128_ref, dc0_ref, dc1_ref,
               now_ref, nob_ref, g1_ref, b1_ref, f1w_ref, f1b_ref, f2w_ref,
               f2b_ref, g2_ref, b2_ref, nh_ref):
    rexp = 1.0 / (s_ref[0] + s_ref[1] + 1e-16)
    ov = ov_ref[0] + jnp.dot(ov_ref[1], w128_ref[...],
                             preferred_element_type=jnp.float32)
    ov = ov * rexp
    nh = ov * (dc0_ref[...] + ld_ref[...] * dc1_ref[...])
    nh = x_ref[...] + jnp.dot(nh, now_ref[...],
                              preferred_element_type=jnp.float32) + nob_ref[...]
    mu = jnp.mean(nh, axis=0, keepdims=True)
    var = jnp.mean((nh - mu) * (nh - mu), axis=0, keepdims=True)
    nh = g1_ref[...] * (nh - mu) / jnp.sqrt(var + 1e-5) + b1_ref[...]
    ff = jax.nn.relu(jnp.dot(nh, f1w_ref[...],
                             preferred_element_type=jnp.float32) + f1b_ref[...])
    ff = jnp.dot(ff, f2w_ref[...],
                 preferred_element_type=jnp.float32) + f2b_ref[...]
    nh = nh + ff
    mu = jnp.mean(nh, axis=0, keepdims=True)
    var = jnp.mean((nh - mu) * (nh - mu), axis=0, keepdims=True)
    nh_ref[...] = g2_ref[...] * (nh - mu) / jnp.sqrt(var + 1e-5) + b2_ref[...]


def _node_tc(ovrv, s2tab, x, ld, w128, dc0, dc1, now, nob, g1, b1, f1w, f1b,
             f2w, f2b, g2, b2):
    return pl.pallas_call(
        _node_body,
        out_shape=jax.ShapeDtypeStruct((N, D), jnp.float32),
    )(ovrv, s2tab, x, ld, w128, dc0, dc1, now, nob, g1, b1, f1w, f1b, f2w,
      f2b, g2, b2)


def _ebn_body(ehp_ref, sums_ref, g_ref, b_ref, eh_ref):
    mu = sums_ref[0:1, :] / E
    m2 = sums_ref[1:2, :] / E
    var = m2 - mu * mu
    scale = g_ref[...] / jnp.sqrt(var + 1e-5)
    eh_ref[...] = (ehp_ref[...] - mu) * scale + b_ref[...]


def _ebn_tc(ehp, sums, g, b):
    return pl.pallas_call(
        _ebn_body,
        grid=(NBLK,),
        in_specs=[
            pl.BlockSpec((BLK, D), lambda i: (i, 0)),
            pl.BlockSpec((8, D), lambda i: (0, 0)),
            pl.BlockSpec((1, D), lambda i: (0, 0)),
            pl.BlockSpec((1, D), lambda i: (0, 0)),
        ],
        out_specs=pl.BlockSpec((BLK, D), lambda i: (i, 0)),
        out_shape=jax.ShapeDtypeStruct((E, D), jnp.float32),
    )(ehp, sums, g, b)


def kernel(x, edge_attr, edge_index, log_deg, Qw, Qb, Kw, Kb, Ew, Eb, Vw, Vb,
           Aw, VeRow, deg_coef, Now, Nob, Eow, Eob, bn1n_g, bn1n_b, bn1e_g,
           bn1e_b, F1w, F1b, F2w, F2b, bn2_g, bn2_b):
    src = edge_index[0]
    dst = edge_index[1]

    wcat = jnp.concatenate([Qw, Kw, Vw], axis=1)
    bcat = jnp.concatenate([Qb, Kb, Vb])[None, :]
    perm = np.concatenate([
        np.arange(H)[:, None] * 2 * HD + np.arange(HD)[None, :],
        np.arange(H)[:, None] * 2 * HD + HD + np.arange(HD)[None, :],
    ]).reshape(2, H * HD).reshape(-1)
    ewp = Ew[:, perm]
    ebp = Eb[perm][None, :]
    a128 = jnp.zeros((D, HD), jnp.float32)
    w128 = jnp.zeros((D, D), jnp.float32)
    for h in range(H):
        a128 = a128.at[h * HD:(h + 1) * HD, h].set(Aw[:, h, 0])
        w128 = w128.at[h * HD:(h + 1) * HD, h * HD:(h + 1) * HD].set(
            VeRow[:, h, :])
    dc0 = deg_coef[0, :, 0][None, :]
    dc1 = deg_coef[0, :, 1][None, :]
    expm = jnp.zeros((HD, D), jnp.float32)
    for h in range(H):
        expm = expm.at[h, h * HD:(h + 1) * HD].set(1.0)
    zeros128 = jnp.zeros((N, D), jnp.float32)

    qh, kh, vh = _qkv(x, wcat, bcat)
    score1 = _score1_sc(kh, qh, src, dst)
    etw, w, w16, ehp, sums = _edge_tc(
        edge_attr, score1, ewp.astype(jnp.bfloat16), ebp,
        a128.astype(jnp.bfloat16), expm, Eow.astype(jnp.bfloat16),
        Eob[None, :])
    s2tab = _stab_sc(w, dst, zeros128)
    ovrv = _agg_sc(vh, etw, w16, src, dst, zeros128)
    nh = _node_tc(ovrv, s2tab, x, log_deg, w128, dc0, dc1, Now, Nob[None, :],
                  bn1n_g[None, :], bn1n_b[None, :], F1w, F1b[None, :],
                  F2w, F2b[None, :], bn2_g[None, :], bn2_b[None, :])
    eh = _ebn_tc(ehp, sums, bn1e_g[None, :], bn1e_b[None, :])
    return nh, eh


# speedup vs baseline: 1.0499x; 1.0499x over previous
import functools

import jax
import jax.numpy as jnp
import numpy as np
from jax import lax
from jax.experimental import pallas as pl
from jax.experimental.pallas import tpu as pltpu
from jax.experimental.pallas import tpu_sc as plsc

N = 10000
E = 320000
D = 128
H = 8
HD = 16

NC = 2
NS = 16
NW = NC * NS

BLK = 2000
NBLK = E // BLK

CH = 40


def _mesh():
    return plsc.VectorSubcoreMesh(core_axis_name="c", subcore_axis_name="s")


def _qkv_body(x_ref, w_ref, b_ref, q_ref, k_ref, v_ref):
    y = jnp.dot(x_ref[...], w_ref[...], preferred_element_type=jnp.float32)
    y = y + b_ref[...]
    q_ref[...] = y[:, :D]
    k_ref[...] = y[:, D:2 * D]
    v_ref[...] = y[:, 2 * D:]


def _qkv(x, wcat, bcat):
    return pl.pallas_call(
        _qkv_body,
        out_shape=[jax.ShapeDtypeStruct((N, D), jnp.float32)] * 3,
    )(x, wcat, bcat)


def _score1_sc(kh, qh, src, dst):
    EP = E // NW
    NB = 5
    CHS1 = 80
    NG = EP // CHS1 // NB
    assert NG * NB * CHS1 == EP

    scratch = []
    scratch += [pltpu.VMEM((CHS1,), jnp.int32) for _ in range(NB)]
    scratch += [pltpu.VMEM((CHS1,), jnp.int32) for _ in range(NB)]
    scratch += [pltpu.VMEM((CHS1, D), jnp.float32) for _ in range(NB)]
    scratch += [pltpu.VMEM((CHS1, D), jnp.float32) for _ in range(NB)]
    scratch += [pltpu.SemaphoreType.DMA for _ in range(3 * NB)]

    @functools.partial(
        pl.kernel,
        mesh=_mesh(),
        out_type=jax.ShapeDtypeStruct((E, D), jnp.float32),
        scratch_types=scratch,
    )
    def k(kh_hbm, qh_hbm, src_hbm, dst_hbm, out_hbm, *refs):
        nb = NB
        isv = refs[0:nb]
        idv = refs[nb:2 * nb]
        bk = refs[2 * nb:3 * nb]
        bq = refs[3 * nb:4 * nb]
        sem_l = refs[4 * nb:5 * nb]
        sem_g = refs[5 * nb:6 * nb]
        sem_o = refs[6 * nb:7 * nb]

        wid = lax.axis_index("s") * NC + lax.axis_index("c")
        tbase = wid * EP

        def issue_lin(g, b):
            sl = pl.ds(tbase + (g * nb + b) * CHS1, CHS1)
            pltpu.async_copy(src_hbm.at[sl], isv[b], sem_l[b])
            pltpu.async_copy(dst_hbm.at[sl], idv[b], sem_l[b])

        def wait_lin(b):
            pltpu.make_async_copy(src_hbm.at[pl.ds(0, CHS1)], isv[b],
                                  sem_l[b]).wait()
            pltpu.make_async_copy(dst_hbm.at[pl.ds(0, CHS1)], idv[b],
                                  sem_l[b]).wait()

        def group(g, carry):
            for b in range(nb):
                wait_lin(b)
                pltpu.async_copy(kh_hbm.at[isv[b]], bk[b], sem_g[b])
                pltpu.async_copy(qh_hbm.at[idv[b]], bq[b], sem_g[b])
            for b in range(nb):
                pltpu.make_async_copy(kh_hbm.at[isv[b]], bk[b],
                                      sem_g[b]).wait()
                pltpu.make_async_copy(qh_hbm.at[idv[b]], bq[b],
                                      sem_g[b]).wait()

                def row(i, carry2, b=b):
                    for j in (2 * i, 2 * i + 1):
                        for h in range(H):
                            sl = pl.ds(h * HD, HD)
                            bk[b][j, sl] = bk[b][j, sl] + bq[b][j, sl]
                    return carry2

                lax.fori_loop(0, CHS1 // 2, row, 0)
                out_sl = pl.ds(tbase + (g * nb + b) * CHS1, CHS1)
                pltpu.async_copy(bk[b], out_hbm.at[out_sl], sem_o[b])
            for b in range(nb):
                pltpu.make_async_copy(bk[b], out_hbm.at[pl.ds(0, CHS1)],
                                      sem_o[b]).wait()

                @pl.when(g < NG - 1)
                def _(b=b):
                    issue_lin(g + 1, b)

            return carry

        for b in range(nb):
            issue_lin(0, b)
        lax.fori_loop(0, NG, group, 0)

    return k(kh, qh, src, dst)


def _edge_body(ea_ref, s1_ref, ew_ref, eb_ref, a_ref, exp_ref, eow_ref,
               eob_ref, etw_ref, w_ref, w16_ref, ehp_ref, sums_ref):
    i = pl.program_id(0)
    ea = ea_ref[...]
    ef = jnp.dot(ea, ew_ref[...], preferred_element_type=jnp.float32)
    ef = ef + eb_ref[...]
    s2 = ef[:, :D] * ef[:, D:]
    pn = jnp.sign(s2) * jnp.sqrt(jnp.abs(s2))
    et = jax.nn.relu(s1_ref[...].astype(jnp.float32) + pn)
    sc = jnp.dot(et, a_ref[...], preferred_element_type=jnp.float32)
    w16 = jnp.exp(jnp.clip(sc, -5.0, 5.0))
    w16_ref[...] = w16
    wx = jnp.dot(w16, exp_ref[...], preferred_element_type=jnp.float32)
    w_ref[...] = wx
    etw_ref[...] = et * wx
    ehp = ea + jnp.dot(et, eow_ref[...], preferred_element_type=jnp.float32)
    ehp = ehp + eob_ref[...]
    ehp_ref[...] = ehp.astype(jnp.bfloat16)

    @pl.when(i == 0)
    def _():
        sums_ref[...] = jnp.zeros_like(sums_ref)

    sums_ref[0:1, :] = sums_ref[0:1, :] + jnp.sum(ehp, axis=0, keepdims=True)
    sums_ref[1:2, :] = sums_ref[1:2, :] + jnp.sum(ehp * ehp, axis=0,
                                                  keepdims=True)


def _edge_tc(edge_attr, score1, ewp, ebp, a128, expm, eow, eob):
    return pl.pallas_call(
        _edge_body,
        grid=(NBLK,),
        in_specs=[
            pl.BlockSpec((BLK, D), lambda i: (i, 0)),
            pl.BlockSpec((BLK, D), lambda i: (i, 0)),
            pl.BlockSpec((D, 2 * D), lambda i: (0, 0)),
            pl.BlockSpec((1, 2 * D), lambda i: (0, 0)),
            pl.BlockSpec((D, HD), lambda i: (0, 0)),
            pl.BlockSpec((HD, D), lambda i: (0, 0)),
            pl.BlockSpec((D, D), lambda i: (0, 0)),
            pl.BlockSpec((1, D), lambda i: (0, 0)),
        ],
        out_specs=[
            pl.BlockSpec((BLK, D), lambda i: (i, 0)),
            pl.BlockSpec((BLK, D), lambda i: (i, 0)),
            pl.BlockSpec((BLK, HD), lambda i: (i, 0)),
            pl.BlockSpec((BLK, D), lambda i: (i, 0)),
            pl.BlockSpec((8, D), lambda i: (0, 0)),
        ],
        out_shape=[
            jax.ShapeDtypeStruct((E, D), jnp.float32),
            jax.ShapeDtypeStruct((E, D), jnp.float32),
            jax.ShapeDtypeStruct((E, HD), jnp.float32),
            jax.ShapeDtypeStruct((E, D), jnp.bfloat16),
            jax.ShapeDtypeStruct((8, D), jnp.float32),
        ],
    )(edge_attr, score1, ewp, ebp, a128, expm, eow, eob)


def _stab_sc(w, dst, zeros128):
    EP = E // NW
    NB = 2
    CHS = 40
    NG = EP // CHS // NB
    assert NG * NB * CHS == EP
    STRIPE = (N // NS) // 8 * 8
    SREM = N - STRIPE * NS

    scratch = [pltpu.VMEM_SHARED((N, D), jnp.float32)]
    scratch += [pltpu.VMEM((CHS,), jnp.int32) for _ in range(NB)]
    scratch += [pltpu.VMEM((CHS, D), jnp.float32) for _ in range(NB)]
    scratch += [pltpu.SemaphoreType.DMA for _ in range(2 * NB)]

    @functools.partial(
        pl.kernel,
        mesh=_mesh(),
        out_type=jax.ShapeDtypeStruct((NC, N, D), jnp.float32),
        scratch_types=scratch,
    )
    def k(w_hbm, dst_hbm, z_hbm, out_hbm, *refs):
        nb = NB
        stab = refs[0]
        idv = refs[1:1 + nb]
        wb = refs[1 + nb:1 + 2 * nb]
        sem_l = refs[1 + 2 * nb:1 + 3 * nb]
        sem_s = refs[1 + 3 * nb:1 + 4 * nb]

        c = lax.axis_index("c")
        sid = lax.axis_index("s")
        tbase = c * (E // NC) + sid * EP

        rows = pl.ds(sid * STRIPE, STRIPE)
        rrem = pl.ds(NS * STRIPE, SREM)
        pltpu.sync_copy(z_hbm.at[rows], stab.at[rows])

        @pl.when(sid == NS - 1)
        def _():
            pltpu.sync_copy(z_hbm.at[rrem], stab.at[rrem])

        plsc.subcore_barrier()

        def issue_lin(g, b):
            sl = pl.ds(tbase + (g * nb + b) * CHS, CHS)
            pltpu.async_copy(dst_hbm.at[sl], idv[b], sem_l[b])
            pltpu.async_copy(w_hbm.at[sl], wb[b], sem_l[b])

        def group(g, carry):
            for b in range(nb):
                pltpu.make_async_copy(dst_hbm.at[pl.ds(0, CHS)], idv[b],
                                      sem_l[b]).wait()
                pltpu.make_async_copy(w_hbm.at[pl.ds(0, CHS)], wb[b],
                                      sem_l[b]).wait()
                pltpu.async_copy(wb[b], stab.at[idv[b]], sem_s[b], add=True)
            for b in range(nb):
                pltpu.make_async_copy(wb[b], stab.at[idv[b]],
                                      sem_s[b]).wait()

                @pl.when(g < NG - 1)
                def _(b=b):
                    issue_lin(g + 1, b)

            return carry

        for b in range(nb):
            issue_lin(0, b)
        lax.fori_loop(0, NG, group, 0)

        plsc.subcore_barrier()
        pltpu.sync_copy(stab.at[rows], out_hbm.at[c, rows])

        @pl.when(sid == NS - 1)
        def _():
            pltpu.sync_copy(stab.at[rrem], out_hbm.at[c, rrem])

    return k(w, dst, zeros128)


AGG_NB = 4


def _agg_sc(vh, etw, w16, src, dst, zeros128):
    EP = E // NS
    NCH = EP // CH
    NG = NCH // AGG_NB
    TAIL = EP - NG * AGG_NB * CH
    STRIPE = (N // NS) // 8 * 8
    SREM = N - STRIPE * NS

    assert TAIL == 0
    scratch = [pltpu.VMEM_SHARED((N, D), jnp.float32)]
    scratch += [pltpu.VMEM((CH,), jnp.int32) for _ in range(AGG_NB)]
    scratch += [pltpu.VMEM((CH,), jnp.int32) for _ in range(AGG_NB)]
    scratch += [pltpu.VMEM((CH, D), jnp.float32) for _ in range(AGG_NB)]
    scratch += [pltpu.VMEM((CH, HD), jnp.float32) for _ in range(AGG_NB)]
    scratch += [pltpu.SemaphoreType.DMA for _ in range(3 * AGG_NB)]

    @functools.partial(
        pl.kernel,
        mesh=_mesh(),
        out_type=jax.ShapeDtypeStruct((NC, N, D), jnp.float32),
        scratch_types=scratch,
    )
    def k(vh_hbm, etw_hbm, w_hbm, src_hbm, dst_hbm, z_hbm, out_hbm, *refs):
        nb = AGG_NB
        tab = refs[0]
        isv = refs[1:1 + nb]
        idv = refs[1 + nb:1 + 2 * nb]
        pb = refs[1 + 2 * nb:1 + 3 * nb]
        wb = refs[1 + 3 * nb:1 + 4 * nb]
        sem_l = refs[1 + 4 * nb:1 + 5 * nb]
        sem_g = refs[1 + 5 * nb:1 + 6 * nb]
        sem_s = refs[1 + 6 * nb:1 + 7 * nb]

        c = lax.axis_index("c")
        sid = lax.axis_index("s")
        tbase = sid * EP

        rows = pl.ds(sid * STRIPE, STRIPE)
        rrem = pl.ds(NS * STRIPE, SREM)
        pltpu.sync_copy(z_hbm.at[rows], tab.at[rows])

        @pl.when(sid == NS - 1)
        def _():
            pltpu.sync_copy(z_hbm.at[rrem], tab.at[rrem])

        plsc.subcore_barrier()

        def issue_lin(g, b):
            base = tbase + (g * nb + b) * CH
            sl = pl.ds(base, CH)
            pltpu.async_copy(dst_hbm.at[sl], idv[b], sem_l[b])

            @pl.when(c == 0)
            def _():
                pltpu.async_copy(w_hbm.at[sl], wb[b], sem_l[b])
                pltpu.async_copy(src_hbm.at[sl], isv[b], sem_l[b])

            @pl.when(c != 0)
            def _():
                pltpu.async_copy(etw_hbm.at[sl], pb[b], sem_l[b])

        def wait_lin(b):
            pltpu.make_async_copy(dst_hbm.at[pl.ds(0, CH)], idv[b],
                                  sem_l[b]).wait()

            @pl.when(c == 0)
            def _():
                pltpu.make_async_copy(w_hbm.at[pl.ds(0, CH)], wb[b],
                                      sem_l[b]).wait()
                pltpu.make_async_copy(src_hbm.at[pl.ds(0, CH)], isv[b],
                                      sem_l[b]).wait()

            @pl.when(c != 0)
            def _():
                pltpu.make_async_copy(etw_hbm.at[pl.ds(0, CH)], pb[b],
                                      sem_l[b]).wait()

        def group(g, carry):
            @pl.when(c == 0)
            def _():
                for b in range(nb):
                    wait_lin(b)
                    pltpu.async_copy(vh_hbm.at[isv[b]], pb[b], sem_g[b])
                for b in range(nb):
                    pltpu.make_async_copy(vh_hbm.at[isv[b]], pb[b],
                                          sem_g[b]).wait()

                    def row(i, carry2, b=b):
                        for j in (2 * i, 2 * i + 1):
                            wv = wb[b][j, :]
                            for h in range(H):
                                av = jnp.full((HD,), wv[h],
                                              dtype=jnp.float32)
                                sl = pl.ds(h * HD, HD)
                                pb[b][j, sl] = pb[b][j, sl] * av
                        return carry2

                    lax.fori_loop(0, CH // 2, row, 0)
                    pltpu.async_copy(pb[b], tab.at[idv[b]], sem_s[b],
                                     add=True)

            @pl.when(c != 0)
            def _():
                for b in range(nb):
                    wait_lin(b)
                    pltpu.async_copy(pb[b], tab.at[idv[b]], sem_s[b],
                                     add=True)

            for b in range(nb):
                pltpu.make_async_copy(pb[b], tab.at[idv[b]], sem_s[b]).wait()

                @pl.when(g < NG - 1)
                def _(b=b):
                    issue_lin(g + 1, b)

            return carry

        for b in range(nb):
            issue_lin(0, b)
        lax.fori_loop(0, NG, group, 0)

        plsc.subcore_barrier()
        pltpu.sync_copy(tab.at[rows], out_hbm.at[c, rows])

        @pl.when(sid == NS - 1)
        def _():
            pltpu.sync_copy(tab.at[rrem], out_hbm.at[c, rrem])

    return k(vh, etw, w16, src, dst, zeros128)


def _node_body(ov_ref, s_ref, x_ref, ld_ref, w128_ref, dc0_ref, dc1_ref,
               now_ref, nob_ref, g1_ref, b1_ref, f1w_ref, f1b_ref, f2w_ref,
               f2b_ref, g2_ref, b2_ref, nh_ref):
    rexp = 1.0 / (s_ref[0] + s_ref[1] + 1e-16)
    ov = ov_ref[0] + jnp.dot(ov_ref[1], w128_ref[...],
                             preferred_element_type=jnp.float32)
    ov = ov * rexp
    nh = ov * (dc0_ref[...] + ld_ref[...] * dc1_ref[...])
    nh = x_ref[...] + jnp.dot(nh, now_ref[...],
                              preferred_element_type=jnp.float32) + nob_ref[...]
    mu = jnp.mean(nh, axis=0, keepdims=True)
    var = jnp.mean((nh - mu) * (nh - mu), axis=0, keepdims=True)
    nh = g1_ref[...] * (nh - mu) / jnp.sqrt(var + 1e-5) + b1_ref[...]
    ff = jax.nn.relu(jnp.dot(nh, f1w_ref[...],
                             preferred_element_type=jnp.float32) + f1b_ref[...])
    ff = jnp.dot(ff, f2w_ref[...],
                 preferred_element_type=jnp.float32) + f2b_ref[...]
    nh = nh + ff
    mu = jnp.mean(nh, axis=0, keepdims=True)
    var = jnp.mean((nh - mu) * (nh - mu), axis=0, keepdims=True)
    nh_ref[...] = g2_ref[...] * (nh - mu) / jnp.sqrt(var + 1e-5) + b2_ref[...]


def _node_tc(ovrv, s2tab, x, ld, w128, dc0, dc1, now, nob, g1, b1, f1w, f1b,
             f2w, f2b, g2, b2):
    return pl.pallas_call(
        _node_body,
        out_shape=jax.ShapeDtypeStruct((N, D), jnp.float32),
    )(ovrv, s2tab, x, ld, w128, dc0, dc1, now, nob, g1, b1, f1w, f1b, f2w,
      f2b, g2, b2)


def _ebn_body(ehp_ref, sums_ref, g_ref, b_ref, eh_ref):
    mu = sums_ref[0:1, :] / E
    m2 = sums_ref[1:2, :] / E
    var = m2 - mu * mu
    scale = g_ref[...] / jnp.sqrt(var + 1e-5)
    eh_ref[...] = (ehp_ref[...].astype(jnp.float32) - mu) * scale + b_ref[...]


def _ebn_tc(ehp, sums, g, b):
    return pl.pallas_call(
        _ebn_body,
        grid=(NBLK,),
        in_specs=[
            pl.BlockSpec((BLK, D), lambda i: (i, 0)),
            pl.BlockSpec((8, D), lambda i: (0, 0)),
            pl.BlockSpec((1, D), lambda i: (0, 0)),
            pl.BlockSpec((1, D), lambda i: (0, 0)),
        ],
        out_specs=pl.BlockSpec((BLK, D), lambda i: (i, 0)),
        out_shape=jax.ShapeDtypeStruct((E, D), jnp.float32),
    )(ehp, sums, g, b)


def kernel(x, edge_attr, edge_index, log_deg, Qw, Qb, Kw, Kb, Ew, Eb, Vw, Vb,
           Aw, VeRow, deg_coef, Now, Nob, Eow, Eob, bn1n_g, bn1n_b, bn1e_g,
           bn1e_b, F1w, F1b, F2w, F2b, bn2_g, bn2_b):
    src = edge_index[0]
    dst = edge_index[1]

    wcat = jnp.concatenate([Qw, Kw, Vw], axis=1)
    bcat = jnp.concatenate([Qb, Kb, Vb])[None, :]
    perm = np.concatenate([
        np.arange(H)[:, None] * 2 * HD + np.arange(HD)[None, :],
        np.arange(H)[:, None] * 2 * HD + HD + np.arange(HD)[None, :],
    ]).reshape(2, H * HD).reshape(-1)
    ewp = Ew[:, perm]
    ebp = Eb[perm][None, :]
    a128 = jnp.zeros((D, HD), jnp.float32)
    w128 = jnp.zeros((D, D), jnp.float32)
    for h in range(H):
        a128 = a128.at[h * HD:(h + 1) * HD, h].set(Aw[:, h, 0])
        w128 = w128.at[h * HD:(h + 1) * HD, h * HD:(h + 1) * HD].set(
            VeRow[:, h, :])
    dc0 = deg_coef[0, :, 0][None, :]
    dc1 = deg_coef[0, :, 1][None, :]
    expm = jnp.zeros((HD, D), jnp.float32)
    for h in range(H):
        expm = expm.at[h, h * HD:(h + 1) * HD].set(1.0)
    zeros128 = jnp.zeros((N, D), jnp.float32)

    qh, kh, vh = _qkv(x, wcat, bcat)
    score1 = _score1_sc(kh, qh, src, dst)
    etw, w, w16, ehp, sums = _edge_tc(edge_attr, score1, ewp, ebp, a128,
                                      expm, Eow, Eob[None, :])
    s2tab = _stab_sc(w, dst, zeros128)
    ovrv = _agg_sc(vh, etw, w16, src, dst, zeros128)
    nh = _node_tc(ovrv, s2tab, x, log_deg, w128, dc0, dc1, Now, Nob[None, :],
                  bn1n_g[None, :], bn1n_b[None, :], F1w, F1b[None, :],
                  F2w, F2b[None, :], bn2_g[None, :], bn2_b[None, :])
    eh = _ebn_tc(ehp, sums, bn1e_g[None, :], bn1e_b[None, :])
    return nh, eh


# speedup vs baseline: 1.0690x; 1.0182x over previous
import functools

import jax
import jax.numpy as jnp
import numpy as np
from jax import lax
from jax.experimental import pallas as pl
from jax.experimental.pallas import tpu as pltpu
from jax.experimental.pallas import tpu_sc as plsc

N = 10000
E = 320000
D = 128
H = 8
HD = 16

NC = 2
NS = 16
NW = NC * NS

BLK = 2000
NBLK = E // BLK

CH = 40


def _mesh():
    return plsc.VectorSubcoreMesh(core_axis_name="c", subcore_axis_name="s")


def _qkv_body(x_ref, w_ref, b_ref, q_ref, k_ref, v_ref):
    y = jnp.dot(x_ref[...], w_ref[...], preferred_element_type=jnp.float32)
    y = y + b_ref[...]
    q_ref[...] = y[:, :D]
    k_ref[...] = y[:, D:2 * D]
    v_ref[...] = y[:, 2 * D:]


def _qkv(x, wcat, bcat):
    return pl.pallas_call(
        _qkv_body,
        out_shape=[jax.ShapeDtypeStruct((N, D), jnp.float32)] * 3,
    )(x, wcat, bcat)


def _score1_sc(kh, qh, src, dst):
    EP = E // NW
    NB = 5
    CHS1 = 80
    NG = EP // CHS1 // NB
    assert NG * NB * CHS1 == EP

    scratch = []
    scratch += [pltpu.VMEM((CHS1,), jnp.int32) for _ in range(NB)]
    scratch += [pltpu.VMEM((CHS1,), jnp.int32) for _ in range(NB)]
    scratch += [pltpu.VMEM((CHS1, D), jnp.float32) for _ in range(NB)]
    scratch += [pltpu.VMEM((CHS1, D), jnp.float32) for _ in range(NB)]
    scratch += [pltpu.SemaphoreType.DMA for _ in range(3 * NB)]

    @functools.partial(
        pl.kernel,
        mesh=_mesh(),
        out_type=jax.ShapeDtypeStruct((E, D), jnp.float32),
        scratch_types=scratch,
    )
    def k(kh_hbm, qh_hbm, src_hbm, dst_hbm, out_hbm, *refs):
        nb = NB
        isv = refs[0:nb]
        idv = refs[nb:2 * nb]
        bk = refs[2 * nb:3 * nb]
        bq = refs[3 * nb:4 * nb]
        sem_l = refs[4 * nb:5 * nb]
        sem_g = refs[5 * nb:6 * nb]
        sem_o = refs[6 * nb:7 * nb]

        wid = lax.axis_index("s") * NC + lax.axis_index("c")
        tbase = wid * EP

        def issue_lin(g, b):
            sl = pl.ds(tbase + (g * nb + b) * CHS1, CHS1)
            pltpu.async_copy(src_hbm.at[sl], isv[b], sem_l[b])
            pltpu.async_copy(dst_hbm.at[sl], idv[b], sem_l[b])

        def wait_lin(b):
            pltpu.make_async_copy(src_hbm.at[pl.ds(0, CHS1)], isv[b],
                                  sem_l[b]).wait()
            pltpu.make_async_copy(dst_hbm.at[pl.ds(0, CHS1)], idv[b],
                                  sem_l[b]).wait()

        def group(g, carry):
            for b in range(nb):
                wait_lin(b)
                pltpu.async_copy(kh_hbm.at[isv[b]], bk[b], sem_g[b])
                pltpu.async_copy(qh_hbm.at[idv[b]], bq[b], sem_g[b])
            for b in range(nb):
                pltpu.make_async_copy(kh_hbm.at[isv[b]], bk[b],
                                      sem_g[b]).wait()
                pltpu.make_async_copy(qh_hbm.at[idv[b]], bq[b],
                                      sem_g[b]).wait()

                def row(i, carry2, b=b):
                    for j in (2 * i, 2 * i + 1):
                        for h in range(H):
                            sl = pl.ds(h * HD, HD)
                            bk[b][j, sl] = bk[b][j, sl] + bq[b][j, sl]
                    return carry2

                lax.fori_loop(0, CHS1 // 2, row, 0)
                out_sl = pl.ds(tbase + (g * nb + b) * CHS1, CHS1)
                pltpu.async_copy(bk[b], out_hbm.at[out_sl], sem_o[b])
            for b in range(nb):
                pltpu.make_async_copy(bk[b], out_hbm.at[pl.ds(0, CHS1)],
                                      sem_o[b]).wait()

                @pl.when(g < NG - 1)
                def _(b=b):
                    issue_lin(g + 1, b)

            return carry

        for b in range(nb):
            issue_lin(0, b)
        lax.fori_loop(0, NG, group, 0)

    return k(kh, qh, src, dst)


def _edge_body(ea_ref, s1_ref, ew_ref, eb_ref, a_ref, exp_ref, eow_ref,
               eob_ref, etw_ref, w16_ref, ehp_ref, sums_ref):
    i = pl.program_id(0)
    ea = ea_ref[...]
    ef = jnp.dot(ea, ew_ref[...], preferred_element_type=jnp.float32)
    ef = ef + eb_ref[...]
    s2 = ef[:, :D] * ef[:, D:]
    pn = jnp.sign(s2) * jnp.sqrt(jnp.abs(s2))
    et = jax.nn.relu(s1_ref[...].astype(jnp.float32) + pn)
    sc = jnp.dot(et, a_ref[...], preferred_element_type=jnp.float32)
    w16 = jnp.exp(jnp.clip(sc, -5.0, 5.0))
    w16_ref[...] = w16
    wx = jnp.dot(w16, exp_ref[...], preferred_element_type=jnp.float32)
    etw_ref[...] = et * wx
    ehp = ea + jnp.dot(et, eow_ref[...], preferred_element_type=jnp.float32)
    ehp = ehp + eob_ref[...]
    ehp_ref[...] = ehp.astype(jnp.bfloat16)

    @pl.when(i == 0)
    def _():
        sums_ref[...] = jnp.zeros_like(sums_ref)

    sums_ref[0:1, :] = sums_ref[0:1, :] + jnp.sum(ehp, axis=0, keepdims=True)
    sums_ref[1:2, :] = sums_ref[1:2, :] + jnp.sum(ehp * ehp, axis=0,
                                                  keepdims=True)


def _edge_tc(edge_attr, score1, ewp, ebp, a128, expm, eow, eob):
    return pl.pallas_call(
        _edge_body,
        grid=(NBLK,),
        in_specs=[
            pl.BlockSpec((BLK, D), lambda i: (i, 0)),
            pl.BlockSpec((BLK, D), lambda i: (i, 0)),
            pl.BlockSpec((D, 2 * D), lambda i: (0, 0)),
            pl.BlockSpec((1, 2 * D), lambda i: (0, 0)),
            pl.BlockSpec((D, HD), lambda i: (0, 0)),
            pl.BlockSpec((HD, D), lambda i: (0, 0)),
            pl.BlockSpec((D, D), lambda i: (0, 0)),
            pl.BlockSpec((1, D), lambda i: (0, 0)),
        ],
        out_specs=[
            pl.BlockSpec((BLK, D), lambda i: (i, 0)),
            pl.BlockSpec((BLK, HD), lambda i: (i, 0)),
            pl.BlockSpec((BLK, D), lambda i: (i, 0)),
            pl.BlockSpec((8, D), lambda i: (0, 0)),
        ],
        out_shape=[
            jax.ShapeDtypeStruct((E, D), jnp.float32),
            jax.ShapeDtypeStruct((E, HD), jnp.float32),
            jax.ShapeDtypeStruct((E, D), jnp.bfloat16),
            jax.ShapeDtypeStruct((8, D), jnp.float32),
        ],
    )(edge_attr, score1, ewp, ebp, a128, expm, eow, eob)


def _stab_sc(w, dst, zeros128):
    EP = E // NW
    NB = 2
    CHS = 40
    NG = EP // CHS // NB
    assert NG * NB * CHS == EP
    STRIPE = (N // NS) // 8 * 8
    SREM = N - STRIPE * NS

    scratch = [pltpu.VMEM_SHARED((N, D), jnp.float32)]
    scratch += [pltpu.VMEM((CHS,), jnp.int32) for _ in range(NB)]
    scratch += [pltpu.VMEM((CHS, HD), jnp.float32) for _ in range(NB)]
    scratch += [pltpu.VMEM((CHS, D), jnp.float32) for _ in range(NB)]
    scratch += [pltpu.SemaphoreType.DMA for _ in range(2 * NB)]

    @functools.partial(
        pl.kernel,
        mesh=_mesh(),
        out_type=jax.ShapeDtypeStruct((NC, N, D), jnp.float32),
        scratch_types=scratch,
    )
    def k(w_hbm, dst_hbm, z_hbm, out_hbm, *refs):
        nb = NB
        stab = refs[0]
        idv = refs[1:1 + nb]
        wb = refs[1 + nb:1 + 2 * nb]
        px = refs[1 + 2 * nb:1 + 3 * nb]
        sem_l = refs[1 + 3 * nb:1 + 4 * nb]
        sem_s = refs[1 + 4 * nb:1 + 5 * nb]

        c = lax.axis_index("c")
        sid = lax.axis_index("s")
        tbase = c * (E // NC) + sid * EP

        rows = pl.ds(sid * STRIPE, STRIPE)
        rrem = pl.ds(NS * STRIPE, SREM)
        pltpu.sync_copy(z_hbm.at[rows], stab.at[rows])

        @pl.when(sid == NS - 1)
        def _():
            pltpu.sync_copy(z_hbm.at[rrem], stab.at[rrem])

        plsc.subcore_barrier()

        def issue_lin(g, b):
            sl = pl.ds(tbase + (g * nb + b) * CHS, CHS)
            pltpu.async_copy(dst_hbm.at[sl], idv[b], sem_l[b])
            pltpu.async_copy(w_hbm.at[sl], wb[b], sem_l[b])

        def group(g, carry):
            for b in range(nb):
                pltpu.make_async_copy(dst_hbm.at[pl.ds(0, CHS)], idv[b],
                                      sem_l[b]).wait()
                pltpu.make_async_copy(w_hbm.at[pl.ds(0, CHS)], wb[b],
                                      sem_l[b]).wait()

                def row(i, carry2, b=b):
                    for j in (2 * i, 2 * i + 1):
                        wv = wb[b][j, :]
                        for h in range(H):
                            px[b][j, pl.ds(h * HD, HD)] = jnp.full(
                                (HD,), wv[h], dtype=jnp.float32)
                    return carry2

                lax.fori_loop(0, CHS // 2, row, 0)
                pltpu.async_copy(px[b], stab.at[idv[b]], sem_s[b], add=True)
            for b in range(nb):
                pltpu.make_async_copy(px[b], stab.at[idv[b]],
                                      sem_s[b]).wait()

                @pl.when(g < NG - 1)
                def _(b=b):
                    issue_lin(g + 1, b)

            return carry

        for b in range(nb):
            issue_lin(0, b)
        lax.fori_loop(0, NG, group, 0)

        plsc.subcore_barrier()
        pltpu.sync_copy(stab.at[rows], out_hbm.at[c, rows])

        @pl.when(sid == NS - 1)
        def _():
            pltpu.sync_copy(stab.at[rrem], out_hbm.at[c, rrem])

    return k(w, dst, zeros128)


AGG_NB = 4


def _agg_sc(vh, etw, w16, src, dst, zeros128):
    EP = E // NS
    NCH = EP // CH
    NG = NCH // AGG_NB
    TAIL = EP - NG * AGG_NB * CH
    STRIPE = (N // NS) // 8 * 8
    SREM = N - STRIPE * NS

    assert TAIL == 0
    scratch = [pltpu.VMEM_SHARED((N, D), jnp.float32)]
    scratch += [pltpu.VMEM((CH,), jnp.int32) for _ in range(AGG_NB)]
    scratch += [pltpu.VMEM((CH,), jnp.int32) for _ in range(AGG_NB)]
    scratch += [pltpu.VMEM((CH, D), jnp.float32) for _ in range(AGG_NB)]
    scratch += [pltpu.VMEM((CH, HD), jnp.float32) for _ in range(AGG_NB)]
    scratch += [pltpu.SemaphoreType.DMA for _ in range(3 * AGG_NB)]

    @functools.partial(
        pl.kernel,
        mesh=_mesh(),
        out_type=jax.ShapeDtypeStruct((NC, N, D), jnp.float32),
        scratch_types=scratch,
    )
    def k(vh_hbm, etw_hbm, w_hbm, src_hbm, dst_hbm, z_hbm, out_hbm, *refs):
        nb = AGG_NB
        tab = refs[0]
        isv = refs[1:1 + nb]
        idv = refs[1 + nb:1 + 2 * nb]
        pb = refs[1 + 2 * nb:1 + 3 * nb]
        wb = refs[1 + 3 * nb:1 + 4 * nb]
        sem_l = refs[1 + 4 * nb:1 + 5 * nb]
        sem_g = refs[1 + 5 * nb:1 + 6 * nb]
        sem_s = refs[1 + 6 * nb:1 + 7 * nb]

        c = lax.axis_index("c")
        sid = lax.axis_index("s")
        tbase = sid * EP

        rows = pl.ds(sid * STRIPE, STRIPE)
        rrem = pl.ds(NS * STRIPE, SREM)
        pltpu.sync_copy(z_hbm.at[rows], tab.at[rows])

        @pl.when(sid == NS - 1)
        def _():
            pltpu.sync_copy(z_hbm.at[rrem], tab.at[rrem])

        plsc.subcore_barrier()

        def issue_lin(g, b):
            base = tbase + (g * nb + b) * CH
            sl = pl.ds(base, CH)
            pltpu.async_copy(dst_hbm.at[sl], idv[b], sem_l[b])

            @pl.when(c == 0)
            def _():
                pltpu.async_copy(w_hbm.at[sl], wb[b], sem_l[b])
                pltpu.async_copy(src_hbm.at[sl], isv[b], sem_l[b])

            @pl.when(c != 0)
            def _():
                pltpu.async_copy(etw_hbm.at[sl], pb[b], sem_l[b])

        def wait_lin(b):
            pltpu.make_async_copy(dst_hbm.at[pl.ds(0, CH)], idv[b],
                                  sem_l[b]).wait()

            @pl.when(c == 0)
            def _():
                pltpu.make_async_copy(w_hbm.at[pl.ds(0, CH)], wb[b],
                                      sem_l[b]).wait()
                pltpu.make_async_copy(src_hbm.at[pl.ds(0, CH)], isv[b],
                                      sem_l[b]).wait()

            @pl.when(c != 0)
            def _():
                pltpu.make_async_copy(etw_hbm.at[pl.ds(0, CH)], pb[b],
                                      sem_l[b]).wait()

        def group(g, carry):
            @pl.when(c == 0)
            def _():
                for b in range(nb):
                    wait_lin(b)
                    pltpu.async_copy(vh_hbm.at[isv[b]], pb[b], sem_g[b])
                for b in range(nb):
                    pltpu.make_async_copy(vh_hbm.at[isv[b]], pb[b],
                                          sem_g[b]).wait()

                    def row(i, carry2, b=b):
                        for j in (2 * i, 2 * i + 1):
                            wv = wb[b][j, :]
                            for h in range(H):
                                av = jnp.full((HD,), wv[h],
                                              dtype=jnp.float32)
                                sl = pl.ds(h * HD, HD)
                                pb[b][j, sl] = pb[b][j, sl] * av
                        return carry2

                    lax.fori_loop(0, CH // 2, row, 0)
                    pltpu.async_copy(pb[b], tab.at[idv[b]], sem_s[b],
                                     add=True)

            @pl.when(c != 0)
            def _():
                for b in range(nb):
                    wait_lin(b)
                    pltpu.async_copy(pb[b], tab.at[idv[b]], sem_s[b],
                                     add=True)

            for b in range(nb):
                pltpu.make_async_copy(pb[b], tab.at[idv[b]], sem_s[b]).wait()

                @pl.when(g < NG - 1)
                def _(b=b):
                    issue_lin(g + 1, b)

            return carry

        for b in range(nb):
            issue_lin(0, b)
        lax.fori_loop(0, NG, group, 0)

        plsc.subcore_barrier()
        pltpu.sync_copy(tab.at[rows], out_hbm.at[c, rows])

        @pl.when(sid == NS - 1)
        def _():
            pltpu.sync_copy(tab.at[rrem], out_hbm.at[c, rrem])

    return k(vh, etw, w16, src, dst, zeros128)


def _node_body(ov_ref, s_ref, x_ref, ld_ref, w128_ref, dc0_ref, dc1_ref,
               now_ref, nob_ref, g1_ref, b1_ref, f1w_ref, f1b_ref, f2w_ref,
               f2b_ref, g2_ref, b2_ref, nh_ref):
    rexp = 1.0 / (s_ref[0] + s_ref[1] + 1e-16)
    ov = ov_ref[0] + jnp.dot(ov_ref[1], w128_ref[...],
                             preferred_element_type=jnp.float32)
    ov = ov * rexp
    nh = ov * (dc0_ref[...] + ld_ref[...] * dc1_ref[...])
    nh = x_ref[...] + jnp.dot(nh, now_ref[...],
                              preferred_element_type=jnp.float32) + nob_ref[...]
    mu = jnp.mean(nh, axis=0, keepdims=True)
    var = jnp.mean((nh - mu) * (nh - mu), axis=0, keepdims=True)
    nh = g1_ref[...] * (nh - mu) / jnp.sqrt(var + 1e-5) + b1_ref[...]
    ff = jax.nn.relu(jnp.dot(nh, f1w_ref[...],
                             preferred_element_type=jnp.float32) + f1b_ref[...])
    ff = jnp.dot(ff, f2w_ref[...],
                 preferred_element_type=jnp.float32) + f2b_ref[...]
    nh = nh + ff
    mu = jnp.mean(nh, axis=0, keepdims=True)
    var = jnp.mean((nh - mu) * (nh - mu), axis=0, keepdims=True)
    nh_ref[...] = g2_ref[...] * (nh - mu) / jnp.sqrt(var + 1e-5) + b2_ref[...]


def _node_tc(ovrv, s2tab, x, ld, w128, dc0, dc1, now, nob, g1, b1, f1w, f1b,
             f2w, f2b, g2, b2):
    return pl.pallas_call(
        _node_body,
        out_shape=jax.ShapeDtypeStruct((N, D), jnp.float32),
    )(ovrv, s2tab, x, ld, w128, dc0, dc1, now, nob, g1, b1, f1w, f1b, f2w,
      f2b, g2, b2)


def _ebn_body(ehp_ref, sums_ref, g_ref, b_ref, eh_ref):
    mu = sums_ref[0:1, :] / E
    m2 = sums_ref[1:2, :] / E
    var = m2 - mu * mu
    scale = g_ref[...] / jnp.sqrt(var + 1e-5)
    eh_ref[...] = (ehp_ref[...].astype(jnp.float32) - mu) * scale + b_ref[...]


def _ebn_tc(ehp, sums, g, b):
    return pl.pallas_call(
        _ebn_body,
        grid=(NBLK,),
        in_specs=[
            pl.BlockSpec((BLK, D), lambda i: (i, 0)),
            pl.BlockSpec((8, D), lambda i: (0, 0)),
            pl.BlockSpec((1, D), lambda i: (0, 0)),
            pl.BlockSpec((1, D), lambda i: (0, 0)),
        ],
        out_specs=pl.BlockSpec((BLK, D), lambda i: (i, 0)),
        out_shape=jax.ShapeDtypeStruct((E, D), jnp.float32),
    )(ehp, sums, g, b)


def kernel(x, edge_attr, edge_index, log_deg, Qw, Qb, Kw, Kb, Ew, Eb, Vw, Vb,
           Aw, VeRow, deg_coef, Now, Nob, Eow, Eob, bn1n_g, bn1n_b, bn1e_g,
           bn1e_b, F1w, F1b, F2w, F2b, bn2_g, bn2_b):
    src = edge_index[0]
    dst = edge_index[1]

    wcat = jnp.concatenate([Qw, Kw, Vw], axis=1)
    bcat = jnp.concatenate([Qb, Kb, Vb])[None, :]
    perm = np.concatenate([
        np.arange(H)[:, None] * 2 * HD + np.arange(HD)[None, :],
        np.arange(H)[:, None] * 2 * HD + HD + np.arange(HD)[None, :],
    ]).reshape(2, H * HD).reshape(-1)
    ewp = Ew[:, perm]
    ebp = Eb[perm][None, :]
    a128 = jnp.zeros((D, HD), jnp.float32)
    w128 = jnp.zeros((D, D), jnp.float32)
    for h in range(H):
        a128 = a128.at[h * HD:(h + 1) * HD, h].set(Aw[:, h, 0])
        w128 = w128.at[h * HD:(h + 1) * HD, h * HD:(h + 1) * HD].set(
            VeRow[:, h, :])
    dc0 = deg_coef[0, :, 0][None, :]
    dc1 = deg_coef[0, :, 1][None, :]
    expm = jnp.zeros((HD, D), jnp.float32)
    for h in range(H):
        expm = expm.at[h, h * HD:(h + 1) * HD].set(1.0)
    zeros128 = jnp.zeros((N, D), jnp.float32)

    qh, kh, vh = _qkv(x, wcat, bcat)
    score1 = _score1_sc(kh, qh, src, dst)
    etw, w16, ehp, sums = _edge_tc(edge_attr, score1, ewp, ebp, a128,
                                   expm, Eow, Eob[None, :])
    s2tab = _stab_sc(w16, dst, zeros128)
    ovrv = _agg_sc(vh, etw, w16, src, dst, zeros128)
    nh = _node_tc(ovrv, s2tab, x, log_deg, w128, dc0, dc1, Now, Nob[None, :],
                  bn1n_g[None, :], bn1n_b[None, :], F1w, F1b[None, :],
                  F2w, F2b[None, :], bn2_g[None, :], bn2_b[None, :])
    eh = _ebn_tc(ehp, sums, bn1e_g[None, :], bn1e_b[None, :])
    return nh, eh
